# baseline (speedup 1.0000x reference)
import jax
import jax.numpy as jnp
from jax.experimental import pallas as pl
from jax.experimental.pallas import tpu as pltpu

EMB = 768
H = 8
D = 96
E = 8
S = 128
ES = E * S
A = 2
TN = 1024


def _fused(x_ref, wmh_ref, wmg_ref, router_ref, k_ref, v_ref, exp_ref, out_ref):
    xt = x_ref[...]
    ht = jnp.dot(xt, wmh_ref[...].T, preferred_element_type=jnp.float32)
    wmg = wmg_ref[0]
    logits = jnp.dot(ht, router_ref[0], preferred_element_type=jnp.float32)
    probs = jax.nn.softmax(logits, axis=-1)
    cols = jax.lax.broadcasted_iota(jnp.int32, (TN, E), 1)
    i1 = jnp.argmax(probs, axis=-1)
    p1 = jnp.max(probs, axis=-1)
    rest = jnp.where(cols == i1[:, None], -1.0, probs)
    i2 = jnp.argmax(rest, axis=-1)
    p2 = jnp.max(rest, axis=-1)

    hs = (ht * 1.4426950408889634).astype(jnp.bfloat16)
    scores = jnp.dot(hs, k_ref[0].T,
                     preferred_element_type=jnp.float32)
    slot_e = jax.lax.broadcasted_iota(jnp.int32, (TN, ES), 1) // S
    gate = (jnp.where(slot_e == i1[:, None], p1[:, None], 0.0)
            + jnp.where(slot_e == i2[:, None], p2[:, None], 0.0))
    hidden = ((jnp.exp2(scores) - 1.0) * gate).astype(jnp.bfloat16)
    oh = jnp.dot(hidden, v_ref[0], preferred_element_type=jnp.float32)
    contrib = jnp.dot(oh, wmg.T, preferred_element_type=jnp.float32)

    @pl.when(pl.program_id(1) == 0)
    def _init():
        out_ref[...] = contrib

    @pl.when(pl.program_id(1) != 0)
    def _acc():
        out_ref[...] += contrib


def kernel(x, W_mh, W_mg, router, K, V):
    B, T, emb = x.shape
    N = B * T
    x2 = x.reshape(N, emb)
    wmg_r = W_mg.reshape(emb, H, D).transpose(1, 0, 2)
    expand = jnp.kron(jnp.eye(E, dtype=jnp.float32),
                      jnp.ones((1, S), jnp.float32))
    out = pl.pallas_call(
        _fused,
        grid=(N // TN, H),
        in_specs=[
            pl.BlockSpec((TN, EMB), lambda t, h: (t, 0)),
            pl.BlockSpec((D, EMB), lambda t, h: (h, 0)),
            pl.BlockSpec((1, EMB, D), lambda t, h: (h, 0, 0)),
            pl.BlockSpec((1, D, E), lambda t, h: (h, 0, 0)),
            pl.BlockSpec((1, ES, D), lambda t, h: (h, 0, 0)),
            pl.BlockSpec((1, ES, D), lambda t, h: (h, 0, 0)),
            pl.BlockSpec((E, ES), lambda t, h: (0, 0)),
        ],
        out_specs=pl.BlockSpec((TN, EMB), lambda t, h: (t, 0)),
        out_shape=jax.ShapeDtypeStruct((N, EMB), jnp.float32),
        compiler_params=pltpu.CompilerParams(
            dimension_semantics=("parallel", "arbitrary"),
        ),
    )(x2, W_mh, wmg_r, router, K.astype(jnp.bfloat16), V.astype(jnp.bfloat16),
      expand)
    return out.reshape(B, T, emb)


# speedup vs baseline: 1.4753x; 1.4753x over previous
import jax
import jax.numpy as jnp
from jax.experimental import pallas as pl
from jax.experimental.pallas import tpu as pltpu

EMB = 768
H = 8
D = 96
E = 8
S = 128
ES = E * S
A = 2
TN = 1024


def _fused(x_ref, wmh_ref, wmg_ref, router_ref, k_ref, v_ref, out_ref):
    xt = x_ref[...]
    ht_all = jnp.dot(xt, wmh_ref[...].T, preferred_element_type=jnp.float32)
    cols = jax.lax.broadcasted_iota(jnp.int32, (TN, E), 1)
    ohs = []
    for h in range(H):
        ht = ht_all[:, h * D:(h + 1) * D]
        logits = jnp.dot(ht, router_ref[h], preferred_element_type=jnp.float32)
        probs = jax.nn.softmax(logits, axis=-1)
        i1 = jnp.argmax(probs, axis=-1)
        p1 = jnp.max(probs, axis=-1)
        rest = jnp.where(cols == i1[:, None], -1.0, probs)
        i2 = jnp.argmax(rest, axis=-1)
        p2 = jnp.max(rest, axis=-1)

        hs = (ht * 1.4426950408889634).astype(jnp.bfloat16)
        scores = jnp.dot(hs, k_ref[h].T,
                         preferred_element_type=jnp.float32)
        slot_e = jax.lax.broadcasted_iota(jnp.int32, (TN, ES), 1) // S
        gate = (jnp.where(slot_e == i1[:, None], p1[:, None], 0.0)
                + jnp.where(slot_e == i2[:, None], p2[:, None], 0.0))
        hidden = ((jnp.exp2(scores) - 1.0) * gate).astype(jnp.bfloat16)
        ohs.append(jnp.dot(hidden, v_ref[h], preferred_element_type=jnp.float32))
    o_cat = jnp.concatenate(ohs, axis=1)
    out_ref[...] = jnp.dot(o_cat, wmg_ref[...].T, preferred_element_type=jnp.float32)


def kernel(x, W_mh, W_mg, router, K, V):
    B, T, emb = x.shape
    N = B * T
    x2 = x.reshape(N, emb)
    out = pl.pallas_call(
        _fused,
        grid=(N // TN,),
        in_specs=[
            pl.BlockSpec((TN, EMB), lambda t: (t, 0)),
            pl.BlockSpec((H * D, EMB), lambda t: (0, 0)),
            pl.BlockSpec((EMB, H * D), lambda t: (0, 0)),
            pl.BlockSpec((H, D, E), lambda t: (0, 0, 0)),
            pl.BlockSpec((H, ES, D), lambda t: (0, 0, 0)),
            pl.BlockSpec((H, ES, D), lambda t: (0, 0, 0)),
        ],
        out_specs=pl.BlockSpec((TN, EMB), lambda t: (t, 0)),
        out_shape=jax.ShapeDtypeStruct((N, EMB), jnp.float32),
        compiler_params=pltpu.CompilerParams(
            dimension_semantics=("parallel",),
        ),
    )(x2, W_mh, W_mg, router, K.astype(jnp.bfloat16), V.astype(jnp.bfloat16))
    return out.reshape(B, T, emb)


# speedup vs baseline: 1.4796x; 1.0029x over previous
import jax
import jax.numpy as jnp
from jax.experimental import pallas as pl
from jax.experimental.pallas import tpu as pltpu

EMB = 768
H = 8
D = 96
E = 8
S = 128
ES = E * S
A = 2
TN = 512


def _fused(x_ref, wmh_ref, wmg_ref, router_ref, k_ref, v_ref, out_ref):
    xt = x_ref[...]
    ht_all = jnp.dot(xt, wmh_ref[...].T, preferred_element_type=jnp.float32)
    cols = jax.lax.broadcasted_iota(jnp.int32, (TN, E), 1)
    ohs = []
    for h in range(H):
        ht = ht_all[:, h * D:(h + 1) * D]
        logits = jnp.dot(ht, router_ref[h], preferred_element_type=jnp.float32)
        probs = jax.nn.softmax(logits, axis=-1)
        i1 = jnp.argmax(probs, axis=-1)
        p1 = jnp.max(probs, axis=-1)
        rest = jnp.where(cols == i1[:, None], -1.0, probs)
        i2 = jnp.argmax(rest, axis=-1)
        p2 = jnp.max(rest, axis=-1)

        hs = (ht * 1.4426950408889634).astype(jnp.bfloat16)
        scores = jnp.dot(hs, k_ref[h].T,
                         preferred_element_type=jnp.float32)
        slot_e = jax.lax.broadcasted_iota(jnp.int32, (TN, ES), 1) // S
        gate = (jnp.where(slot_e == i1[:, None], p1[:, None], 0.0)
                + jnp.where(slot_e == i2[:, None], p2[:, None], 0.0))
        hidden = ((jnp.exp2(scores) - 1.0) * gate).astype(jnp.bfloat16)
        ohs.append(jnp.dot(hidden, v_ref[h], preferred_element_type=jnp.float32))
    o_cat = jnp.concatenate(ohs, axis=1)
    out_ref[...] = jnp.dot(o_cat, wmg_ref[...].T, preferred_element_type=jnp.float32)


def kernel(x, W_mh, W_mg, router, K, V):
    B, T, emb = x.shape
    N = B * T
    x2 = x.reshape(N, emb)
    out = pl.pallas_call(
        _fused,
        grid=(N // TN,),
        in_specs=[
            pl.BlockSpec((TN, EMB), lambda t: (t, 0)),
            pl.BlockSpec((H * D, EMB), lambda t: (0, 0)),
            pl.BlockSpec((EMB, H * D), lambda t: (0, 0)),
            pl.BlockSpec((H, D, E), lambda t: (0, 0, 0)),
            pl.BlockSpec((H, ES, D), lambda t: (0, 0, 0)),
            pl.BlockSpec((H, ES, D), lambda t: (0, 0, 0)),
        ],
        out_specs=pl.BlockSpec((TN, EMB), lambda t: (t, 0)),
        out_shape=jax.ShapeDtypeStruct((N, EMB), jnp.float32),
        compiler_params=pltpu.CompilerParams(
            dimension_semantics=("parallel",),
        ),
    )(x2, W_mh, W_mg, router, K.astype(jnp.bfloat16), V.astype(jnp.bfloat16))
    return out.reshape(B, T, emb)


# speedup vs baseline: 1.5732x; 1.0633x over previous
import jax
import jax.numpy as jnp
from jax.experimental import pallas as pl
from jax.experimental.pallas import tpu as pltpu

EMB = 768
H = 8
D = 96
E = 8
S = 128
ES = E * S
A = 2
TN = 512


def _fused(x_ref, wmh_ref, wmg_ref, router_ref, k_ref, v_ref, out_ref):
    xt = x_ref[...]
    ht_all = jnp.dot(xt, wmh_ref[...].T, preferred_element_type=jnp.float32)
    cols = jax.lax.broadcasted_iota(jnp.int32, (TN, E), 1)
    ohs = []
    for h in range(H):
        ht = ht_all[:, h * D:(h + 1) * D]
        logits = jnp.dot(ht, router_ref[h], preferred_element_type=jnp.float32)
        probs = jax.nn.softmax(logits, axis=-1)
        i1 = jnp.argmax(probs, axis=-1)
        p1 = jnp.max(probs, axis=-1)
        rest = jnp.where(cols == i1[:, None], -1.0, probs)
        i2 = jnp.argmax(rest, axis=-1)
        p2 = jnp.max(rest, axis=-1)

        hs = (ht * 1.4426950408889634).astype(jnp.bfloat16)
        scores = jnp.dot(hs, k_ref[h].T,
                         preferred_element_type=jnp.float32)
        slot_e = (jax.lax.broadcasted_iota(jnp.int32, (TN, ES), 1)
                  // S).astype(jnp.bfloat16)
        zb = jnp.zeros((), jnp.bfloat16)
        i1b = i1.astype(jnp.bfloat16)
        i2b = i2.astype(jnp.bfloat16)
        p1b = p1.astype(jnp.bfloat16)
        p2b = p2.astype(jnp.bfloat16)
        gate = (jnp.where(slot_e == i1b[:, None], p1b[:, None], zb)
                + jnp.where(slot_e == i2b[:, None], p2b[:, None], zb))
        em1 = (jnp.exp2(scores) - 1.0).astype(jnp.bfloat16)
        hidden = em1 * gate
        ohs.append(jnp.dot(hidden, v_ref[h], preferred_element_type=jnp.float32))
    o_cat = jnp.concatenate(ohs, axis=1)
    out_ref[...] = jnp.dot(o_cat, wmg_ref[...].T, preferred_element_type=jnp.float32)


def kernel(x, W_mh, W_mg, router, K, V):
    B, T, emb = x.shape
    N = B * T
    x2 = x.reshape(N, emb)
    out = pl.pallas_call(
        _fused,
        grid=(N // TN,),
        in_specs=[
            pl.BlockSpec((TN, EMB), lambda t: (t, 0)),
            pl.BlockSpec((H * D, EMB), lambda t: (0, 0)),
            pl.BlockSpec((EMB, H * D), lambda t: (0, 0)),
            pl.BlockSpec((H, D, E), lambda t: (0, 0, 0)),
            pl.BlockSpec((H, ES, D), lambda t: (0, 0, 0)),
            pl.BlockSpec((H, ES, D), lambda t: (0, 0, 0)),
        ],
        out_specs=pl.BlockSpec((TN, EMB), lambda t: (t, 0)),
        out_shape=jax.ShapeDtypeStruct((N, EMB), jnp.float32),
        compiler_params=pltpu.CompilerParams(
            dimension_semantics=("parallel",),
        ),
    )(x2, W_mh, W_mg, router, K.astype(jnp.bfloat16), V.astype(jnp.bfloat16))
    return out.reshape(B, T, emb)


# speedup vs baseline: 1.7074x; 1.0853x over previous
import jax
import jax.numpy as jnp
from jax.experimental import pallas as pl
from jax.experimental.pallas import tpu as pltpu

EMB = 768
H = 8
D = 96
E = 8
S = 128
ES = E * S
A = 2
TN = 512


def _fused(x_ref, wmh_ref, wmg_ref, router_ref, k_ref, v_ref, exp_ref, out_ref):
    xt = x_ref[...]
    ht_all = jnp.dot(xt, wmh_ref[...].T, preferred_element_type=jnp.float32)
    ohs = []
    for h in range(H):
        ht = ht_all[:, h * D:(h + 1) * D]
        logits = jnp.dot(ht, router_ref[h], preferred_element_type=jnp.float32)
        m1 = jnp.max(logits, axis=-1, keepdims=True)
        m2 = jnp.max(jnp.where(logits < m1, logits, -jnp.inf),
                     axis=-1, keepdims=True)
        el = jnp.exp2((logits - m1) * 1.4426950408889634)
        r = 1.0 / jnp.sum(el, axis=-1, keepdims=True)
        gv = jnp.where(logits >= m2, el * r, 0.0).astype(jnp.bfloat16)

        hs = (ht * 1.4426950408889634).astype(jnp.bfloat16)
        scores = jnp.dot(hs, k_ref[h].T,
                         preferred_element_type=jnp.float32)
        gate = jnp.dot(gv, exp_ref[...],
                       preferred_element_type=jnp.float32
                       ).astype(jnp.bfloat16)
        em1 = (jnp.exp2(scores) - 1.0).astype(jnp.bfloat16)
        hidden = em1 * gate
        ohs.append(jnp.dot(hidden, v_ref[h], preferred_element_type=jnp.float32))
    o_cat = jnp.concatenate(ohs, axis=1)
    out_ref[...] = jnp.dot(o_cat, wmg_ref[...].T, preferred_element_type=jnp.float32)


def kernel(x, W_mh, W_mg, router, K, V):
    B, T, emb = x.shape
    N = B * T
    x2 = x.reshape(N, emb)
    out_call = pl.pallas_call(
        _fused,
        grid=(N // TN,),
        in_specs=[
            pl.BlockSpec((TN, EMB), lambda t: (t, 0)),
            pl.BlockSpec((H * D, EMB), lambda t: (0, 0)),
            pl.BlockSpec((EMB, H * D), lambda t: (0, 0)),
            pl.BlockSpec((H, D, E), lambda t: (0, 0, 0)),
            pl.BlockSpec((H, ES, D), lambda t: (0, 0, 0)),
            pl.BlockSpec((H, ES, D), lambda t: (0, 0, 0)),
            pl.BlockSpec((E, ES), lambda t: (0, 0)),
        ],
        out_specs=pl.BlockSpec((TN, EMB), lambda t: (t, 0)),
        out_shape=jax.ShapeDtypeStruct((N, EMB), jnp.float32),
        compiler_params=pltpu.CompilerParams(
            dimension_semantics=("parallel",),
        ),
    )
    expand = jnp.kron(jnp.eye(E, dtype=jnp.bfloat16),
                      jnp.ones((1, S), jnp.bfloat16))
    out = out_call(x2, W_mh, W_mg, router, K.astype(jnp.bfloat16),
                   V.astype(jnp.bfloat16), expand)
    return out.reshape(B, T, emb)


# speedup vs baseline: 1.7275x; 1.0118x over previous
import jax
import jax.numpy as jnp
from jax.experimental import pallas as pl
from jax.experimental.pallas import tpu as pltpu

EMB = 768
H = 8
D = 96
E = 8
S = 128
ES = E * S
A = 2
TN = 1024


def _fused(x_ref, wmh_ref, wmg_ref, router_ref, k_ref, v_ref, exp_ref, out_ref):
    xt = x_ref[...]
    ht_all = jnp.dot(xt, wmh_ref[...].T, preferred_element_type=jnp.float32)
    ohs = []
    for h in range(H):
        ht = ht_all[:, h * D:(h + 1) * D]
        logits = jnp.dot(ht, router_ref[h], preferred_element_type=jnp.float32)
        m1 = jnp.max(logits, axis=-1, keepdims=True)
        m2 = jnp.max(jnp.where(logits < m1, logits, -jnp.inf),
                     axis=-1, keepdims=True)
        el = jnp.exp2((logits - m1) * 1.4426950408889634)
        r = 1.0 / jnp.sum(el, axis=-1, keepdims=True)
        gv = jnp.where(logits >= m2, el * r, 0.0).astype(jnp.bfloat16)

        hs = (ht * 1.4426950408889634).astype(jnp.bfloat16)
        scores = jnp.dot(hs, k_ref[h].T,
                         preferred_element_type=jnp.float32)
        gate = jnp.dot(gv, exp_ref[...],
                       preferred_element_type=jnp.float32
                       ).astype(jnp.bfloat16)
        em1 = (jnp.exp2(scores) - 1.0).astype(jnp.bfloat16)
        hidden = em1 * gate
        ohs.append(jnp.dot(hidden, v_ref[h], preferred_element_type=jnp.float32))
    o_cat = jnp.concatenate(ohs, axis=1)
    out_ref[...] = jnp.dot(o_cat, wmg_ref[...].T, preferred_element_type=jnp.float32)


def kernel(x, W_mh, W_mg, router, K, V):
    B, T, emb = x.shape
    N = B * T
    x2 = x.reshape(N, emb)
    out_call = pl.pallas_call(
        _fused,
        grid=(N // TN,),
        in_specs=[
            pl.BlockSpec((TN, EMB), lambda t: (t, 0)),
            pl.BlockSpec((H * D, EMB), lambda t: (0, 0)),
            pl.BlockSpec((EMB, H * D), lambda t: (0, 0)),
            pl.BlockSpec((H, D, E), lambda t: (0, 0, 0)),
            pl.BlockSpec((H, ES, D), lambda t: (0, 0, 0)),
            pl.BlockSpec((H, ES, D), lambda t: (0, 0, 0)),
            pl.BlockSpec((E, ES), lambda t: (0, 0)),
        ],
        out_specs=pl.BlockSpec((TN, EMB), lambda t: (t, 0)),
        out_shape=jax.ShapeDtypeStruct((N, EMB), jnp.float32),
        compiler_params=pltpu.CompilerParams(
            dimension_semantics=("parallel",),
        ),
    )
    expand = jnp.kron(jnp.eye(E, dtype=jnp.bfloat16),
                      jnp.ones((1, S), jnp.bfloat16))
    out = out_call(x2, W_mh, W_mg, router, K.astype(jnp.bfloat16),
                   V.astype(jnp.bfloat16), expand)
    return out.reshape(B, T, emb)
